# baseline (speedup 1.0000x reference)
import functools

import jax
import jax.numpy as jnp
from jax import lax
from jax.experimental import pallas as pl
from jax.experimental.pallas import tpu as pltpu
from jax.experimental.pallas import tpu_sc as plsc

NC = 2
NS = 16
NW = NC * NS

CHUNK = 128
NBUF = 4
PRIME = 2


def _make_sc_gather(total, d):
    per_w = total // NW
    nchunks = per_w // CHUNK
    ngroups = nchunks // NBUF
    mesh = plsc.VectorSubcoreMesh(core_axis_name="c", subcore_axis_name="s")

    @functools.partial(
        pl.kernel,
        mesh=mesh,
        out_type=jax.ShapeDtypeStruct((total, d), jnp.float32),
        scratch_types=[
            pltpu.VMEM((nchunks, CHUNK), jnp.int32),
            pltpu.VMEM((NBUF, CHUNK, d), jnp.float32),
            pltpu.VMEM_SHARED((NS, CHUNK, d), jnp.float32),
        ]
        + [pltpu.SemaphoreType.DMA] * (2 * NBUF + 1),
    )
    def gather_kernel(idx_hbm, table_hbm, out_hbm, idx_v, rows_v, shr_v, *sems):
        gsems = sems[:NBUF]
        wsems = sems[NBUF : 2 * NBUF]
        w2sems = sems[2 * NBUF :]
        sid = lax.axis_index("s")
        wid = sid * NC + lax.axis_index("c")
        base = wid * per_w
        pltpu.sync_copy(idx_hbm.at[wid], idx_v)

        for b in range(PRIME):
            pltpu.async_copy(table_hbm.at[idx_v.at[b]], rows_v.at[b], gsems[b])

        def group(jo, carry):
            for b in range(NBUF):
                j = jo * NBUF + b
                pltpu.make_async_copy(
                    table_hbm.at[idx_v.at[j]], rows_v.at[b], gsems[b]
                ).wait()
                if b % 2 == 0:
                    pltpu.async_copy(
                        rows_v.at[b],
                        out_hbm.at[pl.ds(base + j * CHUNK, CHUNK)],
                        wsems[b],
                    )
                else:
                    @pl.when(j >= 3)
                    def _():
                        pltpu.make_async_copy(
                            shr_v.at[sid],
                            out_hbm.at[pl.ds(base, CHUNK)],
                            w2sems[0],
                        ).wait()

                    pltpu.sync_copy(rows_v.at[b], shr_v.at[sid])
                    pltpu.async_copy(
                        shr_v.at[sid],
                        out_hbm.at[pl.ds(base + j * CHUNK, CHUNK)],
                        w2sems[0],
                    )
                jn = j + PRIME
                bn = (b + PRIME) % NBUF

                @pl.when(jn < nchunks)
                def _():
                    if bn % 2 == 0:
                        @pl.when(j >= NBUF - PRIME)
                        def _():
                            pltpu.make_async_copy(
                                rows_v.at[bn],
                                out_hbm.at[pl.ds(base, CHUNK)],
                                wsems[bn],
                            ).wait()

                    pltpu.async_copy(
                        table_hbm.at[idx_v.at[jn]], rows_v.at[bn], gsems[bn]
                    )

            return carry

        lax.fori_loop(0, ngroups, group, 0)

        for b in range(NBUF):
            if b % 2 == 0:
                pltpu.make_async_copy(
                    rows_v.at[b], out_hbm.at[pl.ds(base, CHUNK)], wsems[b]
                ).wait()
        pltpu.make_async_copy(
            shr_v.at[sid], out_hbm.at[pl.ds(base, CHUNK)], w2sems[0]
        ).wait()

    return gather_kernel


def kernel(x, table):
    total = x.shape[0] * x.shape[1]
    d = table.shape[1]
    idx = x.astype(jnp.int32).reshape(NW, total // (NW * CHUNK), CHUNK)
    out = _make_sc_gather(total, d)(idx, table)
    return out.reshape(x.shape[0], x.shape[1], d)


# speedup vs baseline: 1.0318x; 1.0318x over previous
import functools

import jax
import jax.numpy as jnp
from jax import lax
from jax.experimental import pallas as pl
from jax.experimental.pallas import tpu as pltpu
from jax.experimental.pallas import tpu_sc as plsc

NC = 2
NS = 16
NW = NC * NS

CHUNK = 128
NBUF = 5
PRIME = 3


def _make_sc_gather(total, d):
    per_w = total // NW
    nchunks = per_w // CHUNK
    ngroups = nchunks // NBUF
    mesh = plsc.VectorSubcoreMesh(core_axis_name="c", subcore_axis_name="s")

    @functools.partial(
        pl.kernel,
        mesh=mesh,
        out_type=jax.ShapeDtypeStruct((total, d), jnp.float32),
        scratch_types=[
            pltpu.VMEM((nchunks, CHUNK), jnp.int32),
            pltpu.VMEM((NBUF, CHUNK, d), jnp.float32),
            pltpu.VMEM_SHARED((NS, CHUNK, d), jnp.float32),
        ]
        + [pltpu.SemaphoreType.DMA] * (2 * NBUF + 1),
    )
    def gather_kernel(idx_hbm, table_hbm, out_hbm, idx_v, rows_v, shr_v, *sems):
        gsems = sems[:NBUF]
        wsems = sems[NBUF : 2 * NBUF]
        w2sems = sems[2 * NBUF :]
        sid = lax.axis_index("s")
        wid = sid * NC + lax.axis_index("c")
        base = wid * per_w
        pltpu.sync_copy(idx_hbm.at[wid], idx_v)

        for b in range(PRIME):
            pltpu.async_copy(table_hbm.at[idx_v.at[b]], rows_v.at[b], gsems[b])

        def group(jo, carry):
            for b in range(NBUF):
                j = jo * NBUF + b
                pltpu.make_async_copy(
                    table_hbm.at[idx_v.at[j]], rows_v.at[b], gsems[b]
                ).wait()
                if b != 1:
                    pltpu.async_copy(
                        rows_v.at[b],
                        out_hbm.at[pl.ds(base + j * CHUNK, CHUNK)],
                        wsems[b],
                    )
                else:
                    @pl.when(j >= NBUF)
                    def _():
                        pltpu.make_async_copy(
                            shr_v.at[sid],
                            out_hbm.at[pl.ds(base, CHUNK)],
                            w2sems[0],
                        ).wait()

                    pltpu.sync_copy(rows_v.at[b], shr_v.at[sid])
                    pltpu.async_copy(
                        shr_v.at[sid],
                        out_hbm.at[pl.ds(base + j * CHUNK, CHUNK)],
                        w2sems[0],
                    )
                jn = j + PRIME
                bn = (b + PRIME) % NBUF

                @pl.when(jn < nchunks)
                def _():
                    if bn != 1:
                        @pl.when(j >= NBUF - PRIME)
                        def _():
                            pltpu.make_async_copy(
                                rows_v.at[bn],
                                out_hbm.at[pl.ds(base, CHUNK)],
                                wsems[bn],
                            ).wait()

                    pltpu.async_copy(
                        table_hbm.at[idx_v.at[jn]], rows_v.at[bn], gsems[bn]
                    )

            return carry

        lax.fori_loop(0, ngroups, group, 0)

        for b in range(NBUF):
            if b != 1:
                pltpu.make_async_copy(
                    rows_v.at[b], out_hbm.at[pl.ds(base, CHUNK)], wsems[b]
                ).wait()
        pltpu.make_async_copy(
            shr_v.at[sid], out_hbm.at[pl.ds(base, CHUNK)], w2sems[0]
        ).wait()

    return gather_kernel


def kernel(x, table):
    total = x.shape[0] * x.shape[1]
    d = table.shape[1]
    idx = x.astype(jnp.int32).reshape(NW, total // (NW * CHUNK), CHUNK)
    out = _make_sc_gather(total, d)(idx, table)
    return out.reshape(x.shape[0], x.shape[1], d)


# speedup vs baseline: 1.0540x; 1.0216x over previous
import functools

import jax
import jax.numpy as jnp
from jax import lax
from jax.experimental import pallas as pl
from jax.experimental.pallas import tpu as pltpu
from jax.experimental.pallas import tpu_sc as plsc

NC = 2
NS = 16
NW = NC * NS

CHUNK = 128
NBUF = 5
PRIME = 3


def _make_sc_gather(total, d):
    per_w = total // NW
    nchunks = per_w // CHUNK
    ngroups = nchunks // NBUF
    mesh = plsc.VectorSubcoreMesh(core_axis_name="c", subcore_axis_name="s")

    @functools.partial(
        pl.kernel,
        mesh=mesh,
        out_type=jax.ShapeDtypeStruct((total, d), jnp.float32),
        scratch_types=[
            pltpu.VMEM((nchunks, CHUNK), jnp.int32),
            pltpu.VMEM((NBUF, CHUNK, d), jnp.float32),
            pltpu.VMEM_SHARED((NS, CHUNK, d), jnp.float32),
        ]
        + [pltpu.SemaphoreType.DMA] * (2 * NBUF + 2),
    )
    def gather_kernel(idx_hbm, table_hbm, out_hbm, idx_v, rows_v, shr_v, *sems):
        gsems = sems[:NBUF]
        wsems = sems[NBUF : 2 * NBUF]
        w2sems = sems[2 * NBUF : 2 * NBUF + 1]
        hsem = sems[2 * NBUF + 1]
        sid = lax.axis_index("s")
        wid = sid * NC + lax.axis_index("c")
        base = wid * per_w
        pltpu.sync_copy(idx_hbm.at[wid], idx_v)

        for b in range(PRIME):
            pltpu.async_copy(table_hbm.at[idx_v.at[b]], rows_v.at[b], gsems[b])

        def group(jo, carry):
            for b in range(NBUF):
                j = jo * NBUF + b
                pltpu.make_async_copy(
                    table_hbm.at[idx_v.at[j]], rows_v.at[b], gsems[b]
                ).wait()
                if b % 2 == 0:
                    pltpu.async_copy(
                        rows_v.at[b],
                        out_hbm.at[pl.ds(base + j * CHUNK, CHUNK)],
                        wsems[b],
                    )
                    if b > 0:
                        pltpu.make_async_copy(
                            rows_v.at[b - 1], shr_v.at[sid], hsem
                        ).wait()
                        pltpu.async_copy(
                            shr_v.at[sid],
                            out_hbm.at[pl.ds(base + (j - 1) * CHUNK, CHUNK)],
                            w2sems[0],
                        )
                else:
                    cond = j > 4 if b == 1 else j >= 3

                    @pl.when(cond)
                    def _():
                        pltpu.make_async_copy(
                            shr_v.at[sid],
                            out_hbm.at[pl.ds(base, CHUNK)],
                            w2sems[0],
                        ).wait()

                    pltpu.async_copy(rows_v.at[b], shr_v.at[sid], hsem)
                jn = j + PRIME
                bn = (b + PRIME) % NBUF

                @pl.when(jn < nchunks)
                def _():
                    if bn % 2 == 0:
                        @pl.when(j >= NBUF - PRIME)
                        def _():
                            pltpu.make_async_copy(
                                rows_v.at[bn],
                                out_hbm.at[pl.ds(base, CHUNK)],
                                wsems[bn],
                            ).wait()

                    pltpu.async_copy(
                        table_hbm.at[idx_v.at[jn]], rows_v.at[bn], gsems[bn]
                    )

            return carry

        lax.fori_loop(0, ngroups, group, 0)

        for b in range(NBUF):
            if b % 2 == 0:
                pltpu.make_async_copy(
                    rows_v.at[b], out_hbm.at[pl.ds(base, CHUNK)], wsems[b]
                ).wait()
        pltpu.make_async_copy(
            shr_v.at[sid], out_hbm.at[pl.ds(base, CHUNK)], w2sems[0]
        ).wait()

    return gather_kernel


def kernel(x, table):
    total = x.shape[0] * x.shape[1]
    d = table.shape[1]
    idx = x.astype(jnp.int32).reshape(NW, total // (NW * CHUNK), CHUNK)
    out = _make_sc_gather(total, d)(idx, table)
    return out.reshape(x.shape[0], x.shape[1], d)
